# Initial kernel scaffold; baseline (speedup 1.0000x reference)
import jax
import jax.numpy as jnp
from jax.experimental import pallas as pl


def kernel(traj, actions, Wf, bf, Wb, bb, answer):
    raise NotImplementedError("write your pallas kernel here")


# speedup vs baseline: 1.1060x; 1.1060x over previous
import jax
import jax.numpy as jnp
from jax.experimental import pallas as pl
from jax.experimental.pallas import tpu as pltpu


def _block_body(A, traj_ref, w_ref, bias_ref, af_ref, ab_ref, fwd_ref, bwd_ref):
    x = traj_ref[:, :]
    w = w_ref[:, :]
    logits = jnp.dot(x, w, preferred_element_type=jnp.float32)
    logits = logits + bias_ref[:, :]
    R = logits.shape[0]
    ids = jax.lax.broadcasted_iota(jnp.int32, (R, A), 1)

    def select_prob(l, act):
        m = jnp.max(l, axis=1, keepdims=True)
        e = jnp.exp(l - m)
        s = jnp.sum(e, axis=1, keepdims=True)
        sel = jnp.sum(jnp.where(ids == act, e, 0.0), axis=1, keepdims=True)
        return sel / s

    af = af_ref[:, :]
    ab = ab_ref[:, :]
    fwd_ref[:, :] = select_prob(logits[:, :A], af)
    bwd = select_prob(logits[:, A:], ab)
    bwd_ref[:, :] = jnp.where(ab == 2, 1.0, bwd)


def kernel(traj, actions, Wf, bf, Wb, bb, answer):
    B, T, D = traj.shape
    A = Wf.shape[1]
    N = B * T
    ROWS = 1024

    traj2 = traj.reshape(N, D)
    w = jnp.concatenate([Wf, Wb], axis=1)
    bias = jnp.concatenate([bf, bb]).reshape(1, 2 * A)
    acts = actions.reshape(N).astype(jnp.int32)
    af = acts.reshape(N, 1)
    ab = jnp.roll(acts, 1).reshape(N, 1)

    grid = (N // ROWS,)
    fwd, bwd = pl.pallas_call(
        lambda *refs: _block_body(A, *refs),
        grid=grid,
        in_specs=[
            pl.BlockSpec((ROWS, D), lambda i: (i, 0)),
            pl.BlockSpec((D, 2 * A), lambda i: (0, 0)),
            pl.BlockSpec((1, 2 * A), lambda i: (0, 0)),
            pl.BlockSpec((ROWS, 1), lambda i: (i, 0)),
            pl.BlockSpec((ROWS, 1), lambda i: (i, 0)),
        ],
        out_specs=[
            pl.BlockSpec((ROWS, 1), lambda i: (i, 0)),
            pl.BlockSpec((ROWS, 1), lambda i: (i, 0)),
        ],
        out_shape=[
            jax.ShapeDtypeStruct((N, 1), jnp.float32),
            jax.ShapeDtypeStruct((N, 1), jnp.float32),
        ],
        compiler_params=pltpu.CompilerParams(
            dimension_semantics=("arbitrary",),
        ),
    )(traj2, w, bias, af, ab)

    fwd_sel = fwd.reshape(B, T)
    back_sel = bwd.reshape(B, T)[:, 1:]
    rewards = jnp.ones((), jnp.float32)
    return fwd_sel, back_sel, rewards


# speedup vs baseline: 1.5218x; 1.3760x over previous
import jax
import jax.numpy as jnp
from jax.experimental import pallas as pl
from jax.experimental.pallas import tpu as pltpu


def _block_body(A, traj_ref, w_ref, bias_ref, af_ref, ab_ref, fwd_ref, bwd_ref):
    BB, T, D = traj_ref.shape
    x = traj_ref[...].reshape(BB * T, D)
    w = w_ref[:, :]
    logits = jnp.dot(x, w, preferred_element_type=jnp.float32)
    logits = logits + bias_ref[:, :]
    R = logits.shape[0]
    ids = jax.lax.broadcasted_iota(jnp.int32, (R, A), 1)

    def select_prob(l, act):
        m = jnp.max(l, axis=1, keepdims=True)
        e = jnp.exp(l - m)
        s = jnp.sum(e, axis=1, keepdims=True)
        sel = jnp.sum(jnp.where(ids == act, e, 0.0), axis=1, keepdims=True)
        return sel / s

    af = af_ref[:, :]
    ab = ab_ref[:, :]
    fwd_ref[:, :] = select_prob(logits[:, :A], af)
    bwd = select_prob(logits[:, A:], ab)
    bwd_ref[:, :] = jnp.where(ab == 2, 1.0, bwd)


def kernel(traj, actions, Wf, bf, Wb, bb, answer):
    B, T, D = traj.shape
    A = Wf.shape[1]
    N = B * T
    BB = 64
    ROWS = BB * T

    w = jnp.concatenate([Wf, Wb], axis=1)
    bias = jnp.concatenate([bf, bb]).reshape(1, 2 * A)
    acts = actions.reshape(N).astype(jnp.int32)
    af = acts.reshape(N, 1)
    ab = jnp.roll(acts, 1).reshape(N, 1)

    grid = (B // BB,)
    fwd, bwd = pl.pallas_call(
        lambda *refs: _block_body(A, *refs),
        grid=grid,
        in_specs=[
            pl.BlockSpec((BB, T, D), lambda i: (i, 0, 0)),
            pl.BlockSpec((D, 2 * A), lambda i: (0, 0)),
            pl.BlockSpec((1, 2 * A), lambda i: (0, 0)),
            pl.BlockSpec((ROWS, 1), lambda i: (i, 0)),
            pl.BlockSpec((ROWS, 1), lambda i: (i, 0)),
        ],
        out_specs=[
            pl.BlockSpec((ROWS, 1), lambda i: (i, 0)),
            pl.BlockSpec((ROWS, 1), lambda i: (i, 0)),
        ],
        out_shape=[
            jax.ShapeDtypeStruct((N, 1), jnp.float32),
            jax.ShapeDtypeStruct((N, 1), jnp.float32),
        ],
        compiler_params=pltpu.CompilerParams(
            dimension_semantics=("arbitrary",),
        ),
    )(traj, w, bias, af, ab)

    fwd_sel = fwd.reshape(B, T)
    back_sel = bwd.reshape(B, T)[:, 1:]
    rewards = jnp.ones((), jnp.float32)
    return fwd_sel, back_sel, rewards


# speedup vs baseline: 9.2666x; 6.0891x over previous
import jax
import jax.numpy as jnp
from jax.experimental import pallas as pl
from jax.experimental.pallas import tpu as pltpu


def _block_body(A, x_ref, wt_ref, bias_ref, af_ref, ab_ref, fwd_ref, bwd_ref):
    x = x_ref[0]
    wt = wt_ref[:, :]
    logits = jnp.dot(wt, x, preferred_element_type=jnp.float32)
    logits = logits + bias_ref[:, :]
    Bb = logits.shape[1]
    ids = jax.lax.broadcasted_iota(jnp.int32, (A, Bb), 0)

    def select_prob(l, act):
        m = jnp.max(l, axis=0, keepdims=True)
        e = jnp.exp(l - m)
        s = jnp.sum(e, axis=0, keepdims=True)
        sel = jnp.sum(jnp.where(ids == act, e, 0.0), axis=0, keepdims=True)
        return sel / s

    af = af_ref[0]
    ab = ab_ref[0]
    fwd_ref[0] = select_prob(logits[:A, :], af)
    bwd = select_prob(logits[A:, :], ab)
    bwd_ref[0] = jnp.where(ab == 2, 1.0, bwd)


def kernel(traj, actions, Wf, bf, Wb, bb, answer):
    B, T, D = traj.shape
    A = Wf.shape[1]
    Bb = 2048

    xt = traj.transpose(1, 2, 0)
    wt = jnp.concatenate([Wf.T, Wb.T], axis=0)
    bias = jnp.concatenate([bf, bb]).reshape(2 * A, 1)
    actsT = actions.T.astype(jnp.int32)
    af = actsT.reshape(T, 1, B)
    ab = jnp.roll(actsT, 1, axis=0).reshape(T, 1, B)

    grid = (T, B // Bb)
    fwd, bwd = pl.pallas_call(
        lambda *refs: _block_body(A, *refs),
        grid=grid,
        in_specs=[
            pl.BlockSpec((1, D, Bb), lambda t, j: (t, 0, j)),
            pl.BlockSpec((2 * A, D), lambda t, j: (0, 0)),
            pl.BlockSpec((2 * A, 1), lambda t, j: (0, 0)),
            pl.BlockSpec((1, 1, Bb), lambda t, j: (t, 0, j)),
            pl.BlockSpec((1, 1, Bb), lambda t, j: (t, 0, j)),
        ],
        out_specs=[
            pl.BlockSpec((1, 1, Bb), lambda t, j: (t, 0, j)),
            pl.BlockSpec((1, 1, Bb), lambda t, j: (t, 0, j)),
        ],
        out_shape=[
            jax.ShapeDtypeStruct((T, 1, B), jnp.float32),
            jax.ShapeDtypeStruct((T, 1, B), jnp.float32),
        ],
        compiler_params=pltpu.CompilerParams(
            dimension_semantics=("arbitrary", "arbitrary"),
        ),
    )(xt, wt, bias, af, ab)

    fwd_sel = fwd.reshape(T, B).T
    back_sel = bwd.reshape(T, B)[1:].T
    rewards = jnp.ones((), jnp.float32)
    return fwd_sel, back_sel, rewards


# speedup vs baseline: 9.6298x; 1.0392x over previous
import jax
import jax.numpy as jnp
from jax.experimental import pallas as pl
from jax.experimental.pallas import tpu as pltpu


def _block_body(A, T, x_ref, wt_ref, bias_ref, acts_ref, fwd_ref, bwd_ref):
    x = x_ref[0]
    wt = wt_ref[:, :]
    logits = jnp.dot(wt, x, preferred_element_type=jnp.float32)
    logits = logits + bias_ref[:, :]
    Bb = logits.shape[1]
    ids = jax.lax.broadcasted_iota(jnp.int32, (A, Bb), 0)

    def select_prob(l, act):
        m = jnp.max(l, axis=0, keepdims=True)
        e = jnp.exp(l - m)
        s = jnp.sum(e, axis=0, keepdims=True)
        sel = jnp.sum(jnp.where(ids == act, e, 0.0), axis=0, keepdims=True)
        return sel / s

    t = pl.program_id(0)
    af = acts_ref[pl.ds(t, 1), :]
    ab = acts_ref[pl.ds(jax.lax.rem(t + T - 1, T), 1), :]
    fwd_ref[0] = select_prob(logits[:A, :], af)
    bwd = select_prob(logits[A:, :], ab)
    bwd_ref[0] = jnp.where(ab == 2, 1.0, bwd)


def kernel(traj, actions, Wf, bf, Wb, bb, answer):
    B, T, D = traj.shape
    A = Wf.shape[1]
    Bb = 2048

    xt = traj.transpose(1, 2, 0)
    wt = jnp.concatenate([Wf.T, Wb.T], axis=0)
    bias = jnp.concatenate([bf, bb]).reshape(2 * A, 1)
    actsT = actions.T.astype(jnp.int32)

    grid = (T, B // Bb)
    fwd, bwd = pl.pallas_call(
        lambda *refs: _block_body(A, T, *refs),
        grid=grid,
        in_specs=[
            pl.BlockSpec((1, D, Bb), lambda t, j: (t, 0, j)),
            pl.BlockSpec((2 * A, D), lambda t, j: (0, 0)),
            pl.BlockSpec((2 * A, 1), lambda t, j: (0, 0)),
            pl.BlockSpec((T, Bb), lambda t, j: (0, j)),
        ],
        out_specs=[
            pl.BlockSpec((1, 1, Bb), lambda t, j: (t, 0, j)),
            pl.BlockSpec((1, 1, Bb), lambda t, j: (t, 0, j)),
        ],
        out_shape=[
            jax.ShapeDtypeStruct((T, 1, B), jnp.float32),
            jax.ShapeDtypeStruct((T, 1, B), jnp.float32),
        ],
        compiler_params=pltpu.CompilerParams(
            dimension_semantics=("arbitrary", "arbitrary"),
        ),
    )(xt, wt, bias, actsT)

    fwd_sel = fwd.reshape(T, B).T
    back_sel = bwd.reshape(T, B)[1:].T
    rewards = jnp.ones((), jnp.float32)
    return fwd_sel, back_sel, rewards


# speedup vs baseline: 9.6649x; 1.0036x over previous
import jax
import jax.numpy as jnp
from jax.experimental import pallas as pl
from jax.experimental.pallas import tpu as pltpu


def _block_body(A, T, x_ref, wt_ref, bias_ref, acts_ref, fwd_ref, bwd_ref):
    x = x_ref[0]
    wt = wt_ref[:, :]
    logits = jnp.dot(wt, x, preferred_element_type=jnp.float32)
    logits = logits + bias_ref[:, :]
    Bb = logits.shape[1]
    ids = jax.lax.broadcasted_iota(jnp.int32, (A, Bb), 0)

    def select_prob(l, act):
        m = jnp.max(l, axis=0, keepdims=True)
        e = jnp.exp(l - m)
        s = jnp.sum(e, axis=0, keepdims=True)
        sel = jnp.sum(jnp.where(ids == act, e, 0.0), axis=0, keepdims=True)
        return sel / s

    t = pl.program_id(0)
    af = acts_ref[pl.ds(t, 1), :]
    ab = acts_ref[pl.ds(jax.lax.rem(t + T - 1, T), 1), :]
    fwd_ref[0] = select_prob(logits[:A, :], af)
    bwd = select_prob(logits[A:, :], ab)
    bwd_ref[0] = jnp.where(ab == 2, 1.0, bwd)


def kernel(traj, actions, Wf, bf, Wb, bb, answer):
    B, T, D = traj.shape
    A = Wf.shape[1]
    Bb = 4096

    xt = traj.transpose(1, 2, 0)
    wt = jnp.concatenate([Wf.T, Wb.T], axis=0)
    bias = jnp.concatenate([bf, bb]).reshape(2 * A, 1)
    actsT = actions.T.astype(jnp.int32)

    grid = (T, B // Bb)
    fwd, bwd = pl.pallas_call(
        lambda *refs: _block_body(A, T, *refs),
        grid=grid,
        in_specs=[
            pl.BlockSpec((1, D, Bb), lambda t, j: (t, 0, j)),
            pl.BlockSpec((2 * A, D), lambda t, j: (0, 0)),
            pl.BlockSpec((2 * A, 1), lambda t, j: (0, 0)),
            pl.BlockSpec((T, Bb), lambda t, j: (0, j)),
        ],
        out_specs=[
            pl.BlockSpec((1, 1, Bb), lambda t, j: (t, 0, j)),
            pl.BlockSpec((1, 1, Bb), lambda t, j: (t, 0, j)),
        ],
        out_shape=[
            jax.ShapeDtypeStruct((T, 1, B), jnp.float32),
            jax.ShapeDtypeStruct((T, 1, B), jnp.float32),
        ],
        compiler_params=pltpu.CompilerParams(
            dimension_semantics=("arbitrary", "arbitrary"),
        ),
    )(xt, wt, bias, actsT)

    fwd_sel = fwd.reshape(T, B).T
    back_sel = bwd.reshape(T, B)[1:].T
    rewards = jnp.ones((), jnp.float32)
    return fwd_sel, back_sel, rewards


# speedup vs baseline: 9.9877x; 1.0334x over previous
import jax
import jax.numpy as jnp
from jax.experimental import pallas as pl
from jax.experimental.pallas import tpu as pltpu


def _block_body(A, T, x_ref, wf_ref, wb_ref, bias_ref, acts_ref,
                fwd_ref, bwd_ref, wt_s, bias_s):
    t = pl.program_id(0)

    @pl.when(t == 0)
    def _init():
        wt_s[:A, :] = wf_ref[:, :]
        wt_s[A:, :] = wb_ref[:, :]
        bias_s[:, :] = bias_ref[:, :].T

    x = x_ref[0]
    logits = jnp.dot(wt_s[:, :], x, preferred_element_type=jnp.float32)
    logits = logits + bias_s[:, :]
    Bb = logits.shape[1]
    ids = jax.lax.broadcasted_iota(jnp.int32, (A, Bb), 0)

    def select_prob(l, act):
        m = jnp.max(l, axis=0, keepdims=True)
        e = jnp.exp(l - m)
        s = jnp.sum(e, axis=0, keepdims=True)
        sel = jnp.sum(jnp.where(ids == act, e, 0.0), axis=0, keepdims=True)
        return sel / s

    af = acts_ref[pl.ds(t, 1), :]
    ab = acts_ref[pl.ds(jax.lax.rem(t + T - 1, T), 1), :]
    fwd_ref[pl.ds(t, 1), :] = select_prob(logits[:A, :], af)
    bwd = select_prob(logits[A:, :], ab)
    bwd = jnp.where(ab == 2, 1.0, bwd)

    @pl.when(t > 0)
    def _store_bwd():
        bwd_ref[pl.ds(t - 1, 1), :] = bwd


def kernel(traj, actions, Wf, bf, Wb, bb, answer):
    B, T, D = traj.shape
    A = Wf.shape[1]

    xt = traj.transpose(1, 2, 0)
    wf_t = Wf.T
    wb_t = Wb.T
    bias = jnp.concatenate([bf, bb]).reshape(1, 2 * A)
    actsT = actions.T.astype(jnp.int32)

    fwd, bwd = pl.pallas_call(
        lambda *refs: _block_body(A, T, *refs),
        grid=(T,),
        in_specs=[
            pl.BlockSpec((1, D, B), lambda t: (t, 0, 0)),
            pl.BlockSpec((A, D), lambda t: (0, 0)),
            pl.BlockSpec((A, D), lambda t: (0, 0)),
            pl.BlockSpec((1, 2 * A), lambda t: (0, 0)),
            pl.BlockSpec((T, B), lambda t: (0, 0)),
        ],
        out_specs=[
            pl.BlockSpec((T, B), lambda t: (0, 0)),
            pl.BlockSpec((T - 1, B), lambda t: (0, 0)),
        ],
        out_shape=[
            jax.ShapeDtypeStruct((T, B), jnp.float32),
            jax.ShapeDtypeStruct((T - 1, B), jnp.float32),
        ],
        scratch_shapes=[
            pltpu.VMEM((2 * A, D), jnp.float32),
            pltpu.VMEM((2 * A, 1), jnp.float32),
        ],
        compiler_params=pltpu.CompilerParams(
            dimension_semantics=("arbitrary",),
        ),
    )(xt, wf_t, wb_t, bias, actsT)

    fwd_sel = fwd.T
    back_sel = bwd.T
    rewards = jnp.ones((), jnp.float32)
    return fwd_sel, back_sel, rewards
